# Initial kernel scaffold; baseline (speedup 1.0000x reference)
import jax
import jax.numpy as jnp
from jax.experimental import pallas as pl


def kernel(init, p_lsl_edge_index, p_lsl_edge_attr, W, W1, b1, W2, b2):
    raise NotImplementedError("write your pallas kernel here")


# speedup vs baseline: 3.7699x; 3.7699x over previous
import functools

import jax
import jax.numpy as jnp
from jax import lax
from jax.experimental import pallas as pl
from jax.experimental.pallas import tpu as pltpu
from jax.experimental.pallas import tpu_sc as plsc

_L, _S, _D, _E = 10000, 512, 128, 320000
_NC, _NS = 2, 16
_NW = _NC * _NS
_EPW = _E // _NW
_B = 80
_NK = _EPW // _B
_LP = 10240
_RPT = _LP // _NS
_RB = 1000


def _p1_body(w1_ref, b1_ref, w2t_ref, b2_ref, w_ref, out_ref):
    idx = lax.broadcasted_iota(jnp.int32, (_S, 1), 0).astype(jnp.float32)
    hidden = jnp.maximum(idx * w1_ref[...] + b1_ref[...], 0.0)
    proj = jnp.dot(hidden, w2t_ref[...],
                   preferred_element_type=jnp.float32) + b2_ref[...]
    out_ref[...] = jnp.dot(proj, w_ref[...],
                           preferred_element_type=jnp.float32)


def _emb_body(init_ref, p1_ref, out_ref):
    m = (init_ref[...] != 0.0).astype(jnp.float32)
    cnt = jnp.sum(m, axis=1, keepdims=True)
    num = jnp.dot(m, p1_ref[...], preferred_element_type=jnp.float32)
    out_ref[...] = num / jnp.maximum(cnt, 1.0)


def _comb_mm_body(p_ref, w_ref, out_ref):
    h = jnp.maximum(p_ref[0] + p_ref[1], 0.0)
    out_ref[...] = jnp.dot(h, w_ref[...], preferred_element_type=jnp.float32)


def _comb_relu_body(p_ref, out_ref):
    out_ref[...] = jnp.maximum(p_ref[0] + p_ref[1], 0.0)


def _tc_p1(w1r, b1r, w2t, b2r, w):
    return pl.pallas_call(
        _p1_body,
        out_shape=jax.ShapeDtypeStruct((_S, _D), jnp.float32),
    )(w1r, b1r, w2t, b2r, w)


def _tc_emb(init, p1):
    return pl.pallas_call(
        _emb_body,
        grid=(_L // _RB,),
        in_specs=[
            pl.BlockSpec((_RB, _S), lambda i: (i, 0)),
            pl.BlockSpec((_S, _D), lambda i: (0, 0)),
        ],
        out_specs=pl.BlockSpec((_RB, _D), lambda i: (i, 0)),
        out_shape=jax.ShapeDtypeStruct((_L, _D), jnp.float32),
    )(init, p1)


def _tc_combine_mm(parts, w):
    return pl.pallas_call(
        _comb_mm_body,
        grid=(_L // _RB,),
        in_specs=[
            pl.BlockSpec((2, _RB, _D), lambda i: (0, i, 0)),
            pl.BlockSpec((_D, _D), lambda i: (0, 0)),
        ],
        out_specs=pl.BlockSpec((_RB, _D), lambda i: (i, 0)),
        out_shape=jax.ShapeDtypeStruct((_L, _D), jnp.float32),
    )(parts, w)


def _tc_combine_relu(parts):
    return pl.pallas_call(
        _comb_relu_body,
        grid=(_L // _RB,),
        in_specs=[pl.BlockSpec((2, _RB, _D), lambda i: (0, i, 0))],
        out_specs=pl.BlockSpec((_RB, _D), lambda i: (i, 0)),
        out_shape=jax.ShapeDtypeStruct((_L, _D), jnp.float32),
    )(parts)


def _sc_agg_body(t_hbm, row_hbm, col_hbm, attr_hbm, out_hbm,
                 row_v, col_v, attr_v, gath_v, hagg_sh, sem):
    c = lax.axis_index("c")
    s = lax.axis_index("s")
    wid = s * _NC + c
    ebase = wid * _EPW
    rbase = s * _RPT

    def _z(i, _):
        for d in range(_D // 16):
            gath_v[i, pl.ds(d * 16, 16)] = jnp.zeros((16,), jnp.float32)
        return 0
    lax.fori_loop(0, _B, _z, 0)

    def _zc(j, _):
        pltpu.sync_copy(gath_v, hagg_sh.at[pl.ds(rbase + j * _B, _B)])
        return 0
    lax.fori_loop(0, _RPT // _B, _zc, 0)
    plsc.subcore_barrier()

    def _chunk(k, _):
        base = ebase + k * _B
        pltpu.sync_copy(row_hbm.at[pl.ds(base, _B)], row_v)
        pltpu.sync_copy(col_hbm.at[pl.ds(base, _B)], col_v)
        pltpu.sync_copy(attr_hbm.at[pl.ds(base, _B)], attr_v)
        pltpu.async_copy(t_hbm.at[row_v], gath_v, sem).wait()

        def _scale(j, _):
            av16 = attr_v[pl.ds(j * 16, 16)]
            for l in range(16):
                av = jnp.full((16,), av16[l], jnp.float32)
                i = j * 16 + l
                for d in range(_D // 16):
                    sl = pl.ds(d * 16, 16)
                    gath_v[i, sl] = gath_v[i, sl] * av
            return 0
        lax.fori_loop(0, _B // 16, _scale, 0)

        pltpu.sync_copy(gath_v, hagg_sh.at[col_v], add=True)
        return 0
    lax.fori_loop(0, _NK, _chunk, 0)

    plsc.subcore_barrier()
    pltpu.sync_copy(hagg_sh.at[pl.ds(rbase, _RPT)],
                    out_hbm.at[c, pl.ds(rbase, _RPT)])


@functools.partial(jax.jit, static_argnums=())
def _sc_agg(t, row, col, attr):
    mesh = plsc.VectorSubcoreMesh(core_axis_name="c", subcore_axis_name="s")
    f = pl.kernel(
        _sc_agg_body,
        mesh=mesh,
        out_type=jax.ShapeDtypeStruct((_NC, _LP, _D), jnp.float32),
        scratch_types=[
            pltpu.VMEM((_B,), jnp.int32),
            pltpu.VMEM((_B,), jnp.int32),
            pltpu.VMEM((_B,), jnp.float32),
            pltpu.VMEM((_B, _D), jnp.float32),
            pltpu.VMEM_SHARED((_LP, _D), jnp.float32),
            pltpu.SemaphoreType.DMA,
        ],
    )
    return f(t, row, col, attr)


def kernel(init, p_lsl_edge_index, p_lsl_edge_attr, W, W1, b1, W2, b2):
    row = p_lsl_edge_index[0].astype(jnp.int32)
    col = p_lsl_edge_index[1].astype(jnp.int32)
    attr = p_lsl_edge_attr.astype(jnp.float32)

    w1r = W1.T.reshape(1, 16)
    b1r = b1.reshape(1, 16)
    w2t = W2.T.reshape(16, _D)
    b2r = b2.reshape(1, _D)

    p1 = _tc_p1(w1r, b1r, w2t, b2r, W)
    t = _tc_emb(init, p1)
    for layer in range(3):
        parts = _sc_agg(t, row, col, attr)
        if layer < 2:
            t = _tc_combine_mm(parts, W)
        else:
            t = _tc_combine_relu(parts)
    return t
